# baseline (speedup 1.0000x reference)
import functools

import jax
import jax.numpy as jnp
from jax.experimental import pallas as pl


def _router_body(chan_emb_ref, wr1_ref, br1_ref, wr2_ref, br2_ref, w1f_ref,
                 gates_ref, w1sum_ref, *, R, E):
    hidden = jnp.maximum(
        jax.lax.dot_general(
            chan_emb_ref[...], wr1_ref[...],
            (((1,), (0,)), ((), ())), preferred_element_type=jnp.float32,
        ) + br1_ref[...],
        0.0,
    )
    logits = jax.lax.dot_general(
        hidden, wr2_ref[...],
        (((1,), (0,)), ((), ())), preferred_element_type=jnp.float32,
    ) + br2_ref[...]
    m = jnp.max(logits, axis=-1, keepdims=True)
    ex = jnp.exp(logits - m)
    gates = ex / jnp.sum(ex, axis=-1, keepdims=True)
    gates_ref[...] = jnp.concatenate(
        [jnp.broadcast_to(gates[:, e:e + 1], gates.shape[:1] + (R,))
         for e in range(E)], axis=1)
    w1sum_ref[...] = jnp.sum(w1f_ref[...], axis=0, keepdims=True)


def _moe_body(x0, x1, x2, x3, w1f_ref, w2f_ref, gx_ref, w1s_ref, bd_ref,
              o_ref, *, L, BB):
    xrefs = (x0, x1, x2, x3)
    Q = BB // 4
    N = o_ref.shape[2]
    s1_parts, s2_parts = [], []
    for xr in xrefs:
        x2d = xr[...].reshape(Q * L, N)
        sq2d = x2d * x2d
        s1_parts.append(jax.lax.dot_general(
            bd_ref[...], x2d,
            (((1,), (0,)), ((), ())), preferred_element_type=jnp.float32))
        s2_parts.append(jax.lax.dot_general(
            bd_ref[...], sq2d,
            (((1,), (0,)), ((), ())), preferred_element_type=jnp.float32))
    s1g = jnp.concatenate(s1_parts, axis=0)
    s2g = jnp.concatenate(s2_parts, axis=0)
    mean_all = s1g * (1.0 / L)
    var = (s2g - mean_all * s1g) * (1.0 / (L - 1))
    std_all = jnp.sqrt(var) + 1e-6
    mean_t = jnp.transpose(mean_all)
    rstd_t = 1.0 / jnp.transpose(std_all)
    for i in range(BB):
        xb = xrefs[i // Q][i % Q]
        g = jax.lax.dot_general(
            xb, w1f_ref[...],
            (((0,), (0,)), ((), ())), preferred_element_type=jnp.float32,
        )
        hg = ((g - mean_t[:, i:i + 1] * w1s_ref[...])
              * (rstd_t[:, i:i + 1] * gx_ref[...]))
        out_t = jax.lax.dot_general(
            w2f_ref[...], hg,
            (((0,), (1,)), ((), ())), preferred_element_type=jnp.float32,
        )
        o_ref[i] = out_t * std_all[i:i + 1, :] + mean_all[i:i + 1, :]


def kernel(x, chan_emb, Wr1, br1, Wr2, br2, W1, W2):
    B, L, N = x.shape
    E, _, R = W1.shape
    O = W2.shape[2]
    ER = E * R

    w1f = jnp.transpose(W1, (1, 0, 2)).reshape(L, ER)
    w2f = W2.reshape(ER, O)

    gates_ex, w1sum = pl.pallas_call(
        functools.partial(_router_body, R=R, E=E),
        out_shape=(
            jax.ShapeDtypeStruct((N, ER), jnp.float32),
            jax.ShapeDtypeStruct((1, ER), jnp.float32),
        ),
    )(chan_emb, Wr1, br1.reshape(1, -1), Wr2, br2.reshape(1, -1), w1f)

    BB = 8
    Q = BB // 4
    bd = jnp.kron(jnp.eye(Q, dtype=jnp.float32),
                  jnp.ones((1, L), dtype=jnp.float32))
    out = pl.pallas_call(
        functools.partial(_moe_body, L=L, BB=BB),
        grid=(B // BB,),
        in_specs=[
            pl.BlockSpec((Q, L, N), lambda b, j=j: (4 * b + j, 0, 0))
            for j in range(4)
        ] + [
            pl.BlockSpec((L, ER), lambda b: (0, 0)),
            pl.BlockSpec((ER, O), lambda b: (0, 0)),
            pl.BlockSpec((N, ER), lambda b: (0, 0)),
            pl.BlockSpec((1, ER), lambda b: (0, 0)),
            pl.BlockSpec((Q, Q * L), lambda b: (0, 0)),
        ],
        out_specs=pl.BlockSpec((BB, O, N), lambda b: (b, 0, 0)),
        out_shape=jax.ShapeDtypeStruct((B, O, N), jnp.float32),
    )(x, x, x, x, w1f, w2f, gates_ex, w1sum, bd)
    return out


# speedup vs baseline: 1.0784x; 1.0784x over previous
import functools

import jax
import jax.numpy as jnp
from jax.experimental import pallas as pl


def _router_body(chan_emb_ref, wr1_ref, br1_ref, wr2_ref, br2_ref, w1f_ref,
                 gates_ref, w1sum_ref, *, R, E):
    hidden = jnp.maximum(
        jax.lax.dot_general(
            chan_emb_ref[...], wr1_ref[...],
            (((1,), (0,)), ((), ())), preferred_element_type=jnp.float32,
        ) + br1_ref[...],
        0.0,
    )
    logits = jax.lax.dot_general(
        hidden, wr2_ref[...],
        (((1,), (0,)), ((), ())), preferred_element_type=jnp.float32,
    ) + br2_ref[...]
    m = jnp.max(logits, axis=-1, keepdims=True)
    ex = jnp.exp(logits - m)
    gates = ex / jnp.sum(ex, axis=-1, keepdims=True)
    gates_ref[...] = jnp.concatenate(
        [jnp.broadcast_to(gates[:, e:e + 1], gates.shape[:1] + (R,))
         for e in range(E)], axis=1)
    w1sum_ref[...] = jnp.sum(w1f_ref[...], axis=0, keepdims=True)


def _moe_body(x0, x1, x2, x3, w1f_ref, w2f_ref, gx_ref, w1s_ref, o_ref,
              *, L, BB):
    xrefs = (x0, x1, x2, x3)
    Q = BB // 4
    for i in range(BB):
        xb = xrefs[i // Q][i % Q]
        s1 = jnp.sum(xb, axis=0, keepdims=True)
        s2 = jnp.sum(xb * xb, axis=0, keepdims=True)
        mean = s1 * (1.0 / L)
        var = (s2 - mean * s1) * (1.0 / (L - 1))
        std = jnp.sqrt(var) + 1e-6
        rstd = 1.0 / std
        g = jax.lax.dot_general(
            xb, w1f_ref[...],
            (((0,), (0,)), ((), ())), preferred_element_type=jnp.float32,
        )
        mean_c = jnp.transpose(mean)
        rstd_c = jnp.transpose(rstd)
        hg = (g - mean_c * w1s_ref[...]) * (rstd_c * gx_ref[...])
        out_t = jax.lax.dot_general(
            w2f_ref[...], hg,
            (((0,), (1,)), ((), ())), preferred_element_type=jnp.float32,
        )
        o_ref[i] = out_t * std + mean


def kernel(x, chan_emb, Wr1, br1, Wr2, br2, W1, W2):
    B, L, N = x.shape
    E, _, R = W1.shape
    O = W2.shape[2]
    ER = E * R

    w1f = jnp.transpose(W1, (1, 0, 2)).reshape(L, ER)
    w2f = W2.reshape(ER, O)

    gates_ex, w1sum = pl.pallas_call(
        functools.partial(_router_body, R=R, E=E),
        out_shape=(
            jax.ShapeDtypeStruct((N, ER), jnp.float32),
            jax.ShapeDtypeStruct((1, ER), jnp.float32),
        ),
    )(chan_emb, Wr1, br1.reshape(1, -1), Wr2, br2.reshape(1, -1), w1f)

    BB = 8
    Q = BB // 4
    out = pl.pallas_call(
        functools.partial(_moe_body, L=L, BB=BB),
        grid=(B // BB,),
        in_specs=[
            pl.BlockSpec((Q, L, N), lambda b, j=j: (4 * b + j, 0, 0))
            for j in range(4)
        ] + [
            pl.BlockSpec((L, ER), lambda b: (0, 0)),
            pl.BlockSpec((ER, O), lambda b: (0, 0)),
            pl.BlockSpec((N, ER), lambda b: (0, 0)),
            pl.BlockSpec((1, ER), lambda b: (0, 0)),
        ],
        out_specs=pl.BlockSpec((BB, O, N), lambda b: (b, 0, 0)),
        out_shape=jax.ShapeDtypeStruct((B, O, N), jnp.float32),
    )(x, x, x, x, w1f, w2f, gates_ex, w1sum)
    return out
